# Initial kernel scaffold; baseline (speedup 1.0000x reference)
import functools

import jax
import jax.numpy as jnp
from jax import lax
from jax.experimental import pallas as pl
from jax.experimental.pallas import tpu as pltpu
from jax.experimental.pallas import tpu_sc as plsc

N_KP = 16
INPUT_MIN = -1.0
INPUT_MAX = 1.0
STEP = (INPUT_MAX - INPUT_MIN) / (N_KP - 1)
INV_STEP = (N_KP - 1) / (INPUT_MAX - INPUT_MIN)
LANES = 16

N_WORKERS = 32
CHUNK = 16384


def _coef_body(raw_ref, c0_ref, c1_ref):
    raw = raw_ref[...]
    d = jnp.maximum(raw, 0.0) + jnp.log1p(jnp.exp(-jnp.abs(raw)))
    db = jnp.broadcast_to(d, (N_KP, N_KP))
    row = lax.broadcasted_iota(jnp.int32, (N_KP, N_KP), 0)
    col = lax.broadcasted_iota(jnp.int32, (N_KP, N_KP), 1)
    cs = jnp.sum(jnp.where(col <= row, db, 0.0), axis=1, keepdims=True)
    dnext = jnp.sum(jnp.where(col == row + 1, db, 0.0), axis=1, keepdims=True)
    denom = jnp.max(cs) + 1e-6
    y = cs / denom
    ynext = (cs + dnext) / denom
    jf = lax.broadcasted_iota(jnp.float32, (N_KP, 1), 0)
    kx = INPUT_MIN + jf * STEP
    kxn = INPUT_MIN + (jf + 1.0) * STEP
    c1 = (ynext - y) / (kxn - kx + 1e-8)
    c0_ref[...] = y - c1 * kx
    c1_ref[...] = c1


def _coef_tables(keypoint_y_raw):
    c0, c1 = pl.pallas_call(
        _coef_body,
        out_shape=(
            jax.ShapeDtypeStruct((N_KP, 1), jnp.float32),
            jax.ShapeDtypeStruct((N_KP, 1), jnp.float32),
        ),
    )(keypoint_y_raw.reshape(1, N_KP))
    return c0.reshape(N_KP), c1.reshape(N_KP)


def _sc_body(per_worker, n_chunks,
             x_hbm, c0_hbm, c1_hbm, y_hbm, c0_v, c1_v, xb, yb):
    nc = lax.axis_size("c")
    wid = lax.axis_index("s") * nc + lax.axis_index("c")
    base = wid * per_worker
    pltpu.sync_copy(c0_hbm, c0_v)
    pltpu.sync_copy(c1_hbm, c1_v)

    @pl.loop(0, n_chunks)
    def _chunk(c):
        off = base + c * CHUNK
        pltpu.sync_copy(x_hbm.at[pl.ds(off, CHUNK)], xb)

        @functools.partial(plsc.parallel_loop, 0, CHUNK, step=LANES, unroll=8)
        def _vec(i):
            xv = xb[pl.ds(i, LANES)]
            v = jnp.minimum(jnp.maximum(xv, INPUT_MIN), INPUT_MAX)
            u = (v - INPUT_MIN) * INV_STEP
            j = jnp.minimum(u.astype(jnp.int32), N_KP - 2)
            a = plsc.load_gather(c0_v, [j])
            b = plsc.load_gather(c1_v, [j])
            yb[pl.ds(i, LANES)] = a + b * v

        pltpu.sync_copy(yb, y_hbm.at[pl.ds(off, CHUNK)])


def kernel(x, keypoint_y_raw):
    n = x.size
    per_worker = n // N_WORKERS
    n_chunks = per_worker // CHUNK
    c0, c1 = _coef_tables(keypoint_y_raw)

    mesh = plsc.VectorSubcoreMesh(core_axis_name="c", subcore_axis_name="s")
    sc = pl.kernel(
        functools.partial(_sc_body, per_worker, n_chunks),
        out_type=jax.ShapeDtypeStruct((n,), jnp.float32),
        mesh=mesh,
        scratch_types=[
            pltpu.VMEM((N_KP,), jnp.float32),
            pltpu.VMEM((N_KP,), jnp.float32),
            pltpu.VMEM((CHUNK,), jnp.float32),
            pltpu.VMEM((CHUNK,), jnp.float32),
        ],
    )
    return sc(x, c0, c1)


# speedup vs baseline: 9.5551x; 9.5551x over previous
import functools

import jax
import jax.numpy as jnp
from jax import lax
from jax.experimental import pallas as pl
from jax.experimental.pallas import tpu as pltpu
from jax.experimental.pallas import tpu_sc as plsc

N_KP = 16
INPUT_MIN = -1.0
INPUT_MAX = 1.0
STEP = (INPUT_MAX - INPUT_MIN) / (N_KP - 1)
INV_STEP = (N_KP - 1) / (INPUT_MAX - INPUT_MIN)
LANES = 16

N_WORKERS = 32
CHUNK = 16384


def _coef_body(raw_ref, c0_ref, c1_ref):
    raw = raw_ref[...]
    d = jnp.maximum(raw, 0.0) + jnp.log1p(jnp.exp(-jnp.abs(raw)))
    db = jnp.broadcast_to(d, (N_KP, N_KP))
    row = lax.broadcasted_iota(jnp.int32, (N_KP, N_KP), 0)
    col = lax.broadcasted_iota(jnp.int32, (N_KP, N_KP), 1)
    cs = jnp.sum(jnp.where(col <= row, db, 0.0), axis=1, keepdims=True)
    dnext = jnp.sum(jnp.where(col == row + 1, db, 0.0), axis=1, keepdims=True)
    denom = jnp.max(cs) + 1e-6
    y = cs / denom
    ynext = (cs + dnext) / denom
    jf = lax.broadcasted_iota(jnp.int32, (N_KP, 1), 0).astype(jnp.float32)
    kx = INPUT_MIN + jf * STEP
    kxn = INPUT_MIN + (jf + 1.0) * STEP
    c1 = (ynext - y) / (kxn - kx + 1e-8)
    c0_ref[...] = y - c1 * kx
    c1_ref[...] = c1


def _coef_tables(keypoint_y_raw):
    c0, c1 = pl.pallas_call(
        _coef_body,
        out_shape=(
            jax.ShapeDtypeStruct((N_KP, 1), jnp.float32),
            jax.ShapeDtypeStruct((N_KP, 1), jnp.float32),
        ),
    )(keypoint_y_raw.reshape(1, N_KP))
    return c0.reshape(N_KP), c1.reshape(N_KP)


def _sc_body(per_worker, n_chunks,
             x_hbm, c0_hbm, c1_hbm, y_hbm, c0_v, c1_v, xb, yb):
    nc = lax.axis_size("c")
    wid = lax.axis_index("s") * nc + lax.axis_index("c")
    base = wid * per_worker
    pltpu.sync_copy(c0_hbm, c0_v)
    pltpu.sync_copy(c1_hbm, c1_v)

    @pl.loop(0, n_chunks)
    def _chunk(c):
        off = base + c * CHUNK
        pltpu.sync_copy(x_hbm.at[pl.ds(off, CHUNK)], xb)

        @plsc.parallel_loop(0, CHUNK, step=LANES, unroll=8)
        def _vec(i):
            xv = xb[pl.ds(i, LANES)]
            v = jnp.minimum(jnp.maximum(xv, INPUT_MIN), INPUT_MAX)
            u = (v - INPUT_MIN) * INV_STEP
            j = jnp.minimum(u.astype(jnp.int32), N_KP - 2)
            a = plsc.load_gather(c0_v, [j])
            b = plsc.load_gather(c1_v, [j])
            yb[pl.ds(i, LANES)] = a + b * v

        pltpu.sync_copy(yb, y_hbm.at[pl.ds(off, CHUNK)])


def kernel(x, keypoint_y_raw):
    n = x.size
    per_worker = n // N_WORKERS
    n_chunks = per_worker // CHUNK
    c0, c1 = _coef_tables(keypoint_y_raw)

    mesh = plsc.VectorSubcoreMesh(core_axis_name="c", subcore_axis_name="s")
    sc = pl.kernel(
        functools.partial(_sc_body, per_worker, n_chunks),
        out_type=jax.ShapeDtypeStruct((n,), jnp.float32),
        mesh=mesh,
        scratch_types=[
            pltpu.VMEM((N_KP,), jnp.float32),
            pltpu.VMEM((N_KP,), jnp.float32),
            pltpu.VMEM((CHUNK,), jnp.float32),
            pltpu.VMEM((CHUNK,), jnp.float32),
        ],
        compiler_params=pltpu.CompilerParams(needs_layout_passes=False),
    )
    return sc(x, c0, c1)


# speedup vs baseline: 14.9240x; 1.5619x over previous
import functools

import jax
import jax.numpy as jnp
from jax import lax
from jax.experimental import pallas as pl
from jax.experimental.pallas import tpu as pltpu
from jax.experimental.pallas import tpu_sc as plsc

N_KP = 16
INPUT_MIN = -1.0
INPUT_MAX = 1.0
STEP = (INPUT_MAX - INPUT_MIN) / (N_KP - 1)
INV_STEP = (N_KP - 1) / (INPUT_MAX - INPUT_MIN)
LANES = 16

N_WORKERS = 32
CHUNK = 16384


def _coef_body(raw_ref, c0_ref, c1_ref):
    raw = raw_ref[...]
    d = jnp.maximum(raw, 0.0) + jnp.log1p(jnp.exp(-jnp.abs(raw)))
    db = jnp.broadcast_to(d, (N_KP, N_KP))
    row = lax.broadcasted_iota(jnp.int32, (N_KP, N_KP), 0)
    col = lax.broadcasted_iota(jnp.int32, (N_KP, N_KP), 1)
    cs = jnp.sum(jnp.where(col <= row, db, 0.0), axis=1, keepdims=True)
    dnext = jnp.sum(jnp.where(col == row + 1, db, 0.0), axis=1, keepdims=True)
    denom = jnp.max(cs) + 1e-6
    y = cs / denom
    ynext = (cs + dnext) / denom
    jf = lax.broadcasted_iota(jnp.int32, (N_KP, 1), 0).astype(jnp.float32)
    kx = INPUT_MIN + jf * STEP
    kxn = INPUT_MIN + (jf + 1.0) * STEP
    c1 = (ynext - y) / (kxn - kx + 1e-8)
    c0_ref[...] = y - c1 * kx
    c1_ref[...] = c1


def _coef_tables(keypoint_y_raw):
    c0, c1 = pl.pallas_call(
        _coef_body,
        out_shape=(
            jax.ShapeDtypeStruct((N_KP, 1), jnp.float32),
            jax.ShapeDtypeStruct((N_KP, 1), jnp.float32),
        ),
    )(keypoint_y_raw.reshape(1, N_KP))
    return c0.reshape(N_KP), c1.reshape(N_KP)


def _sc_body(per_worker, n_chunks,
             x_hbm, c0_hbm, c1_hbm, y_hbm, c0_v, c1_v,
             xb0, xb1, yb0, yb1, si0, si1, so0, so1):
    nc = lax.axis_size("c")
    wid = lax.axis_index("s") * nc + lax.axis_index("c")
    base = wid * per_worker
    pltpu.sync_copy(c0_hbm, c0_v)
    pltpu.sync_copy(c1_hbm, c1_v)

    xbufs, ybufs = (xb0, xb1), (yb0, yb1)
    sin, sout = (si0, si1), (so0, so1)

    def compute(xb, yb):
        @plsc.parallel_loop(0, CHUNK, step=LANES, unroll=8)
        def _vec(i):
            xv = xb[pl.ds(i, LANES)]
            v = jnp.minimum(jnp.maximum(xv, INPUT_MIN), INPUT_MAX)
            u = (v - INPUT_MIN) * INV_STEP
            j = jnp.minimum(u.astype(jnp.int32), N_KP - 2)
            a = plsc.load_gather(c0_v, [j])
            b = plsc.load_gather(c1_v, [j])
            yb[pl.ds(i, LANES)] = a + b * v

    def start_in(cc, b):
        return pltpu.async_copy(
            x_hbm.at[pl.ds(base + cc * CHUNK, CHUNK)], xbufs[b], sin[b])

    pend_in = {0: start_in(0, 0), 1: start_in(1, 1)}
    pend_out = {}
    for cc in range(n_chunks):
        b = cc & 1
        pend_in.pop(cc).wait()
        if cc >= 2:
            pend_out.pop(cc - 2).wait()
        compute(xbufs[b], ybufs[b])
        pend_out[cc] = pltpu.async_copy(
            ybufs[b], y_hbm.at[pl.ds(base + cc * CHUNK, CHUNK)], sout[b])
        if cc + 2 < n_chunks:
            pend_in[cc + 2] = start_in(cc + 2, b)
    for cc in sorted(pend_out):
        pend_out.pop(cc).wait()


def kernel(x, keypoint_y_raw):
    n = x.size
    per_worker = n // N_WORKERS
    n_chunks = per_worker // CHUNK
    c0, c1 = _coef_tables(keypoint_y_raw)

    mesh = plsc.VectorSubcoreMesh(core_axis_name="c", subcore_axis_name="s")
    sc = pl.kernel(
        functools.partial(_sc_body, per_worker, n_chunks),
        out_type=jax.ShapeDtypeStruct((n,), jnp.float32),
        mesh=mesh,
        scratch_types=[
            pltpu.VMEM((N_KP,), jnp.float32),
            pltpu.VMEM((N_KP,), jnp.float32),
            pltpu.VMEM((CHUNK,), jnp.float32),
            pltpu.VMEM((CHUNK,), jnp.float32),
            pltpu.VMEM((CHUNK,), jnp.float32),
            pltpu.VMEM((CHUNK,), jnp.float32),
            pltpu.SemaphoreType.DMA,
            pltpu.SemaphoreType.DMA,
            pltpu.SemaphoreType.DMA,
            pltpu.SemaphoreType.DMA,
        ],
        compiler_params=pltpu.CompilerParams(needs_layout_passes=False),
    )
    return sc(x, c0, c1)
